# baseline (speedup 1.0000x reference)
import math

import jax
import jax.numpy as jnp
from jax import lax
from jax.experimental import pallas as pl
from jax.experimental.pallas import tpu as pltpu

_B, _N, _C, _E, _H, _NS = 64, 8192, 16, 512, 8, 10
_DH = _E // _H
_NB = 4096
_NT = _N // _NB
_OUTW = 128
_SCALE = 1.0 / math.sqrt(_DH)
_NEG_INF = float("-inf")


def _attn_topk_body(ns_ref, q_ref, emb_ref, mask_ref, wq_ref, wk_ref,
                    bq_ref, bk_ref, idx_ref, w_ref,
                    qs_scr, s_scr, attn_scr, m_scr, z_scr):
    t = pl.program_id(1)

    @pl.when(t == 0)
    def _project_queries():
        q = lax.dot_general(q_ref[...], wq_ref[0],
                            (((1,), (1,)), ((), ())),
                            preferred_element_type=jnp.float32)
        qs_scr[...] = (q + bq_ref[0]) * _SCALE

    k_t = lax.dot_general(emb_ref[...], wk_ref[0],
                          (((1,), (1,)), ((), ())),
                          preferred_element_type=jnp.float32) + bk_ref[0]
    q_all = qs_scr[...]
    col0 = t * _NB
    bias_t = mask_ref[0, :, pl.ds(col0, _NB)]

    for h in range(_H):
        s = lax.dot_general(q_all[:, h * _DH:(h + 1) * _DH],
                            k_t[:, h * _DH:(h + 1) * _DH],
                            (((1,), (1,)), ((), ())),
                            preferred_element_type=jnp.float32)
        s = s + bias_t
        s_scr[pl.ds(h * _B, _B), pl.ds(col0, _NB)] = s
        mt = jnp.max(s, axis=1, keepdims=True)
        m_old = m_scr[h]
        m_new = jnp.where(t == 0, mt, jnp.maximum(m_old, mt))
        e_sum = jnp.sum(jnp.exp(s - m_new), axis=1, keepdims=True)
        z_old = z_scr[h]
        z_scr[h] = jnp.where(t == 0, e_sum,
                             z_old * jnp.exp(m_old - m_new) + e_sum)
        m_scr[h] = m_new

    @pl.when(t == _NT - 1)
    def _finalize():
        sh = [m_scr[h] + jnp.log(z_scr[h] * float(_H)) for h in range(_H)]

        s_count = jnp.sum(
            jnp.where(mask_ref[...] == 0.0, 1.0, 0.0)).astype(jnp.int32)
        kk = jnp.minimum(ns_ref[0, 0], s_count)

        ocols = lax.broadcasted_iota(jnp.int32, (_B, _OUTW), 1)
        i_acc = jnp.full((_B, _OUTW), -1, jnp.int32)
        w_acc = jnp.zeros((_B, _OUTW), jnp.float32)

        for tt in range(_NT):
            acc = jnp.zeros((_B, _NB), jnp.float32)
            for h in range(_H):
                acc = acc + jnp.exp(s_scr[h * _B:(h + 1) * _B,
                                          tt * _NB:(tt + 1) * _NB]
                                    - sh[h])
            attn_scr[:, tt * _NB:(tt + 1) * _NB] = acc

        cols = lax.broadcasted_iota(jnp.int32, (_B, _N), 1)
        a = attn_scr[...]
        for i in range(_NS):
            best_v = jnp.max(a, axis=1, keepdims=True)
            best_i = jnp.argmax(a, axis=1).astype(jnp.int32)[:, None]
            valid = i < kk
            w_acc = jnp.where(ocols == i,
                              jnp.where(valid, best_v, 0.0), w_acc)
            i_acc = jnp.where(ocols == i,
                              jnp.where(valid, best_i, -1), i_acc)
            if i + 1 < _NS:
                a = jnp.where(cols == best_i, _NEG_INF, a)
        idx_ref[0] = i_acc
        w_ref[0] = w_acc


def _run(query_embeddings, all_embeddings, maskT, ns2, wq, wk, bq, bk,
         interpret=False):
    return pl.pallas_call(
        _attn_topk_body,
        grid=(_C, _NT),
        in_specs=[
            pl.BlockSpec(memory_space=pltpu.SMEM),
            pl.BlockSpec((_B, _E), lambda c, t: (0, 0)),
            pl.BlockSpec((_NB, _E), lambda c, t: (t, 0)),
            pl.BlockSpec((1, 1, _N), lambda c, t: (c, 0, 0)),
            pl.BlockSpec((1, _E, _E), lambda c, t: (c, 0, 0)),
            pl.BlockSpec((1, _E, _E), lambda c, t: (c, 0, 0)),
            pl.BlockSpec((1, 1, _E), lambda c, t: (c, 0, 0)),
            pl.BlockSpec((1, 1, _E), lambda c, t: (c, 0, 0)),
        ],
        out_specs=[
            pl.BlockSpec((1, _B, _OUTW), lambda c, t: (c, 0, 0)),
            pl.BlockSpec((1, _B, _OUTW), lambda c, t: (c, 0, 0)),
        ],
        out_shape=[
            jax.ShapeDtypeStruct((_C, _B, _OUTW), jnp.int32),
            jax.ShapeDtypeStruct((_C, _B, _OUTW), jnp.float32),
        ],
        scratch_shapes=[
            pltpu.VMEM((_B, _E), jnp.float32),
            pltpu.VMEM((_H * _B, _N), jnp.float32),
            pltpu.VMEM((_B, _N), jnp.float32),
            pltpu.VMEM((_H, _B, 1), jnp.float32),
            pltpu.VMEM((_H, _B, 1), jnp.float32),
        ],
        compiler_params=pltpu.CompilerParams(
            dimension_semantics=("arbitrary", "arbitrary")),
        interpret=interpret,
    )(ns2, query_embeddings, all_embeddings, maskT, wq, wk, bq, bk)


def kernel(query_embeddings, all_embeddings, label_mask, n_samples,
           in_proj_weight, in_proj_bias):
    wq = in_proj_weight[:, :_E, :]
    wk = in_proj_weight[:, _E:2 * _E, :]
    bq = in_proj_bias[:, :_E].reshape(_C, 1, _E)
    bk = in_proj_bias[:, _E:2 * _E].reshape(_C, 1, _E)
    maskT = ((label_mask.T.astype(jnp.float32) - 1.0)
             * 1e30).reshape(_C, 1, _N)
    ns2 = jnp.asarray(n_samples, jnp.int32).reshape(1, 1)
    idx_pad, w_pad = _run(query_embeddings, all_embeddings, maskT, ns2,
                          wq, wk, bq, bk)
    return (jnp.transpose(idx_pad[:, :, :_NS], (1, 0, 2)),
            jnp.transpose(w_pad[:, :, :_NS], (1, 0, 2)))


# speedup vs baseline: 1.0482x; 1.0482x over previous
import math

import jax
import jax.numpy as jnp
from jax import lax
from jax.experimental import pallas as pl
from jax.experimental.pallas import tpu as pltpu

_B, _N, _C, _E, _H, _NS = 64, 8192, 16, 512, 8, 10
_DH = _E // _H
_NB = 2048
_NT = _N // _NB
_OUTW = 128
_SCALE = 1.0 / math.sqrt(_DH)
_NEG_INF = float("-inf")


def _attn_topk_body(ns_ref, q_ref, emb_ref, mask_ref, wq_ref, wk_ref,
                    bq_ref, bk_ref, idx_ref, w_ref,
                    qs_scr, s_scr, attn_scr, m_scr, z_scr):
    t = pl.program_id(1)

    @pl.when(t == 0)
    def _project_queries():
        q = lax.dot_general(q_ref[...], wq_ref[0],
                            (((1,), (1,)), ((), ())),
                            preferred_element_type=jnp.float32)
        qs_scr[...] = (q + bq_ref[0]) * _SCALE

    k_t = lax.dot_general(emb_ref[...], wk_ref[0],
                          (((1,), (1,)), ((), ())),
                          preferred_element_type=jnp.float32) + bk_ref[0]
    q_all = qs_scr[...]
    col0 = t * _NB
    bias_t = mask_ref[0, :, pl.ds(col0, _NB)]

    for h in range(_H):
        s = lax.dot_general(q_all[:, h * _DH:(h + 1) * _DH],
                            k_t[:, h * _DH:(h + 1) * _DH],
                            (((1,), (1,)), ((), ())),
                            preferred_element_type=jnp.float32)
        s = s + bias_t
        s_scr[pl.ds(h * _B, _B), pl.ds(col0, _NB)] = s
        mt = jnp.max(s, axis=1, keepdims=True)
        m_old = m_scr[h]
        m_new = jnp.where(t == 0, mt, jnp.maximum(m_old, mt))
        e_sum = jnp.sum(jnp.exp(s - m_new), axis=1, keepdims=True)
        z_old = z_scr[h]
        z_scr[h] = jnp.where(t == 0, e_sum,
                             z_old * jnp.exp(m_old - m_new) + e_sum)
        m_scr[h] = m_new

    @pl.when(t == _NT - 1)
    def _finalize():
        sh = [m_scr[h] + jnp.log(z_scr[h] * float(_H)) for h in range(_H)]

        s_count = jnp.sum(
            jnp.where(mask_ref[...] == 0.0, 1.0, 0.0)).astype(jnp.int32)
        kk = jnp.minimum(ns_ref[0, 0], s_count)

        ocols = lax.broadcasted_iota(jnp.int32, (_B, _OUTW), 1)
        i_acc = jnp.full((_B, _OUTW), -1, jnp.int32)
        w_acc = jnp.zeros((_B, _OUTW), jnp.float32)

        for tt in range(_NT):
            acc = jnp.zeros((_B, _NB), jnp.float32)
            for h in range(_H):
                acc = acc + jnp.exp(s_scr[h * _B:(h + 1) * _B,
                                          tt * _NB:(tt + 1) * _NB]
                                    - sh[h])
            attn_scr[:, tt * _NB:(tt + 1) * _NB] = acc

        cols = lax.broadcasted_iota(jnp.int32, (_B, _N), 1)
        a = attn_scr[...]
        for i in range(_NS):
            best_v = jnp.max(a, axis=1, keepdims=True)
            best_i = jnp.argmax(a, axis=1).astype(jnp.int32)[:, None]
            valid = i < kk
            w_acc = jnp.where(ocols == i,
                              jnp.where(valid, best_v, 0.0), w_acc)
            i_acc = jnp.where(ocols == i,
                              jnp.where(valid, best_i, -1), i_acc)
            if i + 1 < _NS:
                a = jnp.where(cols == best_i, _NEG_INF, a)
        idx_ref[0] = i_acc
        w_ref[0] = w_acc


def _run(query_embeddings, all_embeddings, maskT, ns2, wq, wk, bq, bk,
         interpret=False):
    return pl.pallas_call(
        _attn_topk_body,
        grid=(_C, _NT),
        in_specs=[
            pl.BlockSpec(memory_space=pltpu.SMEM),
            pl.BlockSpec((_B, _E), lambda c, t: (0, 0)),
            pl.BlockSpec((_NB, _E), lambda c, t: (t, 0)),
            pl.BlockSpec((1, 1, _N), lambda c, t: (c, 0, 0)),
            pl.BlockSpec((1, _E, _E), lambda c, t: (c, 0, 0)),
            pl.BlockSpec((1, _E, _E), lambda c, t: (c, 0, 0)),
            pl.BlockSpec((1, 1, _E), lambda c, t: (c, 0, 0)),
            pl.BlockSpec((1, 1, _E), lambda c, t: (c, 0, 0)),
        ],
        out_specs=[
            pl.BlockSpec((1, _B, _OUTW), lambda c, t: (c, 0, 0)),
            pl.BlockSpec((1, _B, _OUTW), lambda c, t: (c, 0, 0)),
        ],
        out_shape=[
            jax.ShapeDtypeStruct((_C, _B, _OUTW), jnp.int32),
            jax.ShapeDtypeStruct((_C, _B, _OUTW), jnp.float32),
        ],
        scratch_shapes=[
            pltpu.VMEM((_B, _E), jnp.float32),
            pltpu.VMEM((_H * _B, _N), jnp.float32),
            pltpu.VMEM((_B, _N), jnp.float32),
            pltpu.VMEM((_H, _B, 1), jnp.float32),
            pltpu.VMEM((_H, _B, 1), jnp.float32),
        ],
        compiler_params=pltpu.CompilerParams(
            dimension_semantics=("arbitrary", "arbitrary")),
        interpret=interpret,
    )(ns2, query_embeddings, all_embeddings, maskT, wq, wk, bq, bk)


def kernel(query_embeddings, all_embeddings, label_mask, n_samples,
           in_proj_weight, in_proj_bias):
    wq = in_proj_weight[:, :_E, :]
    wk = in_proj_weight[:, _E:2 * _E, :]
    bq = in_proj_bias[:, :_E].reshape(_C, 1, _E)
    bk = in_proj_bias[:, _E:2 * _E].reshape(_C, 1, _E)
    maskT = ((label_mask.T.astype(jnp.float32) - 1.0)
             * 1e30).reshape(_C, 1, _N)
    ns2 = jnp.asarray(n_samples, jnp.int32).reshape(1, 1)
    idx_pad, w_pad = _run(query_embeddings, all_embeddings, maskT, ns2,
                          wq, wk, bq, bk)
    return (jnp.transpose(idx_pad[:, :, :_NS], (1, 0, 2)),
            jnp.transpose(w_pad[:, :, :_NS], (1, 0, 2)))
